# baseline (speedup 1.0000x reference)
import functools

import jax
import jax.numpy as jnp
from jax import lax
from jax.experimental import pallas as pl
from jax.experimental.pallas import tpu as pltpu
from jax.experimental.pallas import tpu_sc as plsc

_N, _D, _K = 8192, 1024, 8
_M13 = 0x1FFF
_BR = 256
_BC = 512
_NW = 32
_RPW = _N // _NW
_CB = 8
_NB = _RPW // _CB


def _knn_body(coords_ref, ct_ref, h0_ref, kd_ref, ki_ref, na_ref):
    rx = coords_ref[:, 0:1]
    ry = coords_ref[:, 1:2]
    cx = ct_ref[0:1, :]
    cy = ct_ref[1:2, :]
    col = lax.broadcasted_iota(jnp.int32, (_BR, _N), 1)
    cross = jnp.dot(coords_ref[...], ct_ref[...],
                    preferred_element_type=jnp.float32)
    a2 = rx * rx + ry * ry
    b2 = cx * cx + cy * cy
    d2 = jnp.maximum(a2 + b2 - 2.0 * cross, 0.0)
    key = (lax.bitcast_convert_type(d2, jnp.int32) & ~_M13) | col
    big = jnp.int32(0x7FFFFFFF)
    col8 = lax.broadcasted_iota(jnp.int32, (_BR, _K), 1)
    kd = jnp.zeros((_BR, _K), jnp.float32)
    ki = jnp.zeros((_BR, _K), jnp.int32)
    m = jnp.min(key, axis=1, keepdims=True)
    for k in range(_K):
        m = jnp.min(jnp.where(key > m, key, big), axis=1, keepdims=True)
        idx = m & _M13
        d2t = lax.bitcast_convert_type(m & ~_M13, jnp.float32)
        kd = jnp.where(col8 == k, jnp.sqrt(d2t), kd)
        ki = jnp.where(col8 == k, idx, ki)
    kd_ref[...] = kd
    ki_ref[...] = ki
    h = h0_ref[...]
    na_ref[...] = jnp.sqrt(jnp.sum(h * h, axis=1, keepdims=True))


def _knn_call(coords, ct, h0):
    return pl.pallas_call(
        _knn_body,
        grid=(_N // _BR,),
        in_specs=[
            pl.BlockSpec((_BR, 2), lambda i: (i, 0)),
            pl.BlockSpec((2, _N), lambda i: (0, 0)),
            pl.BlockSpec((_BR, _D), lambda i: (i, 0)),
        ],
        out_specs=[
            pl.BlockSpec((_BR, _K), lambda i: (i, 0)),
            pl.BlockSpec((_BR, _K), lambda i: (i, 0)),
            pl.BlockSpec((_BR, 1), lambda i: (i, 0)),
        ],
        out_shape=[
            jax.ShapeDtypeStruct((_N, _K), jnp.float32),
            jax.ShapeDtypeStruct((_N, _K), jnp.int32),
            jax.ShapeDtypeStruct((_N, 1), jnp.float32),
        ],
    )(coords, ct, h0)


def _sc_dots_body(h0_hbm, ki_hbm, na_hbm, dots_hbm, nna_hbm,
                  na_v, idx_v, self_v, nbr_v, dots_v, nna_v, sem1, sem2):
    wid = lax.axis_index("s") * 2 + lax.axis_index("c")
    base = wid * _RPW
    pltpu.sync_copy(na_hbm, na_v)
    lanes = lax.iota(jnp.int32, 16)

    def batch(b, carry):
        rb = base + b * _CB
        pltpu.sync_copy(ki_hbm.at[pl.ds(rb * _K, _CB * _K)], idx_v)
        cp1 = pltpu.async_copy(h0_hbm.at[idx_v], nbr_v, sem1)
        cp2 = pltpu.async_copy(h0_hbm.at[pl.ds(rb, _CB)], self_v, sem2)
        cp1.wait()
        cp2.wait()
        for p in range(_CB // 2):
            accs = [jnp.zeros((16,), jnp.float32) for _ in range(16)]

            def cchunk(c8, accs):
                accs = list(accs)
                for cc in range(4):
                    off = (c8 * 4 + cc) * 16
                    s0 = self_v[2 * p, pl.ds(off, 16)]
                    s1 = self_v[2 * p + 1, pl.ds(off, 16)]
                    for k in range(_K):
                        accs[k] = accs[k] + s0 * nbr_v[2 * p * _K + k,
                                                       pl.ds(off, 16)]
                        accs[8 + k] = accs[8 + k] + s1 * nbr_v[(2 * p + 1) * _K + k,
                                                               pl.ds(off, 16)]
                return tuple(accs)

            accs = lax.fori_loop(0, _D // 64, cchunk, tuple(accs))
            dvec = jnp.zeros((16,), jnp.float32)
            for j in range(16):
                dvec = jnp.where(lanes == j, jnp.sum(accs[j]), dvec)
            dots_v[pl.ds(p * 16, 16)] = dvec
        for g in range(_CB * _K // 16):
            idx16 = idx_v[pl.ds(g * 16, 16)]
            nna_v[pl.ds(g * 16, 16)] = plsc.load_gather(na_v, [idx16])
        pltpu.sync_copy(dots_v, dots_hbm.at[pl.ds(rb * _K, _CB * _K)])
        pltpu.sync_copy(nna_v, nna_hbm.at[pl.ds(rb * _K, _CB * _K)])
        return carry

    lax.fori_loop(0, _NB, batch, 0)


def _sc_dots(h0, ki_flat, na_flat):
    mesh = plsc.VectorSubcoreMesh(core_axis_name="c", subcore_axis_name="s")
    fn = functools.partial(
        pl.kernel,
        mesh=mesh,
        out_type=[
            jax.ShapeDtypeStruct((_N * _K,), jnp.float32),
            jax.ShapeDtypeStruct((_N * _K,), jnp.float32),
        ],
        scratch_types=[
            pltpu.VMEM((_N,), jnp.float32),
            pltpu.VMEM((_CB * _K,), jnp.int32),
            pltpu.VMEM((_CB, _D), jnp.float32),
            pltpu.VMEM((_CB * _K, _D), jnp.float32),
            pltpu.VMEM((_CB * _K,), jnp.float32),
            pltpu.VMEM((_CB * _K,), jnp.float32),
            pltpu.SemaphoreType.DMA,
            pltpu.SemaphoreType.DMA,
        ],
        compiler_params=pltpu.CompilerParams(needs_layout_passes=False),
    )(_sc_dots_body)
    return fn(h0, ki_flat, na_flat)


def _epi_body(dots_ref, nna_ref, na_ref, kd_ref, w1a_ref, w1b_ref, b1_ref,
              w2r_ref, b2_ref, out_ref):
    na = na_ref[...]
    sim = dots_ref[...] / jnp.maximum(na * nna_ref[...], 1e-8)
    sw = jnp.exp(kd_ref[...] * (-1.0 / 0.05))
    sm = jnp.mean(sim, axis=1, keepdims=True)
    pm = jnp.mean(sw, axis=1, keepdims=True)
    h = jnp.maximum(sm * w1a_ref[...] + pm * w1b_ref[...] + b1_ref[...], 0.0)
    z = jnp.sum(h * w2r_ref[...], axis=1, keepdims=True) + b2_ref[...]
    out_ref[...] = 1.0 / (1.0 + jnp.exp(-z))


def _epi_call(dots, nna, na, kd, w1a, w1b, b1r, w2r, b2r):
    return pl.pallas_call(
        _epi_body,
        grid=(_N // _BC,),
        in_specs=[
            pl.BlockSpec((_BC, _K), lambda i: (i, 0)),
            pl.BlockSpec((_BC, _K), lambda i: (i, 0)),
            pl.BlockSpec((_BC, 1), lambda i: (i, 0)),
            pl.BlockSpec((_BC, _K), lambda i: (i, 0)),
            pl.BlockSpec((1, 64), lambda i: (0, 0)),
            pl.BlockSpec((1, 64), lambda i: (0, 0)),
            pl.BlockSpec((1, 64), lambda i: (0, 0)),
            pl.BlockSpec((1, 64), lambda i: (0, 0)),
            pl.BlockSpec((1, 1), lambda i: (0, 0)),
        ],
        out_specs=pl.BlockSpec((_BC, 1), lambda i: (i, 0)),
        out_shape=jax.ShapeDtypeStruct((_N, 1), jnp.float32),
    )(dots, nna, na, kd, w1a, w1b, b1r, w2r, b2r)


def kernel(h0, coords, W1, b1, W2, b2):
    ct = coords.T
    kd, ki, na = _knn_call(coords, ct, h0)
    dots_f, nna_f = _sc_dots(h0, ki.reshape(-1), na.reshape(-1))
    dots = dots_f.reshape(_N, _K)
    nna = nna_f.reshape(_N, _K)
    w1a = W1[:, 0].reshape(1, 64)
    w1b = W1[:, 1].reshape(1, 64)
    b1r = b1.reshape(1, 64)
    w2r = W2.reshape(1, 64)
    b2r = b2.reshape(1, 1)
    return _epi_call(dots, nna, na, kd, w1a, w1b, b1r, w2r, b2r)


# speedup vs baseline: 1.3554x; 1.3554x over previous
import functools

import jax
import jax.numpy as jnp
from jax import lax
from jax.experimental import pallas as pl
from jax.experimental.pallas import tpu as pltpu
from jax.experimental.pallas import tpu_sc as plsc

_N, _D, _K = 8192, 1024, 8
_M13 = 0x1FFF
_BIAS = 0x00800000
_BR = 256
_BC = 512
_NW = 32
_RPW = _N // _NW
_CB = 4
_NB = _RPW // _CB


def _knn_body(coords_ref, ct_ref, h0_ref, kd_ref, ki_ref, na_ref):
    rx = coords_ref[:, 0:1]
    ry = coords_ref[:, 1:2]
    cx = ct_ref[0:1, :]
    cy = ct_ref[1:2, :]
    col = lax.broadcasted_iota(jnp.int32, (_BR, _N), 1)
    cross = jnp.dot(coords_ref[...], ct_ref[...],
                    preferred_element_type=jnp.float32)
    a2 = rx * rx + ry * ry
    b2 = cx * cx + cy * cy
    d2 = jnp.maximum(a2 + b2 - 2.0 * cross, 0.0)
    keyi = ((lax.bitcast_convert_type(d2, jnp.int32) & ~_M13) | col) + _BIAS
    key = lax.bitcast_convert_type(keyi, jnp.float32)
    big = lax.bitcast_convert_type(jnp.int32(0x7F000000), jnp.float32)
    col8 = lax.broadcasted_iota(jnp.int32, (_BR, _K), 1)
    kd = jnp.zeros((_BR, _K), jnp.float32)
    ki = jnp.zeros((_BR, _K), jnp.int32)
    m = jnp.min(key, axis=1, keepdims=True)
    for k in range(_K):
        m = jnp.min(jnp.where(key > m, key, big), axis=1, keepdims=True)
        mi = lax.bitcast_convert_type(m, jnp.int32) - _BIAS
        idx = mi & _M13
        d2t = lax.bitcast_convert_type(mi & ~_M13, jnp.float32)
        kd = jnp.where(col8 == k, jnp.sqrt(d2t), kd)
        ki = jnp.where(col8 == k, idx, ki)
    kd_ref[...] = kd
    ki_ref[...] = ki
    h = h0_ref[...]
    na_ref[...] = jnp.sqrt(jnp.sum(h * h, axis=1, keepdims=True))


def _knn_call(coords, ct, h0):
    return pl.pallas_call(
        _knn_body,
        grid=(_N // _BR,),
        in_specs=[
            pl.BlockSpec((_BR, 2), lambda i: (i, 0)),
            pl.BlockSpec((2, _N), lambda i: (0, 0)),
            pl.BlockSpec((_BR, _D), lambda i: (i, 0)),
        ],
        out_specs=[
            pl.BlockSpec((_BR, _K), lambda i: (i, 0)),
            pl.BlockSpec((_BR, _K), lambda i: (i, 0)),
            pl.BlockSpec((_BR, 1), lambda i: (i, 0)),
        ],
        out_shape=[
            jax.ShapeDtypeStruct((_N, _K), jnp.float32),
            jax.ShapeDtypeStruct((_N, _K), jnp.int32),
            jax.ShapeDtypeStruct((_N, 1), jnp.float32),
        ],
    )(coords, ct, h0)


def _sc_dots_body(h0_hbm, ki_hbm, na_hbm, dots_hbm, nna_hbm,
                  na_v, idx_v, self_v, nbr_v, dots_v, nna_v, sems, semn):
    wid = lax.axis_index("s") * 2 + lax.axis_index("c")
    base = wid * _RPW
    pltpu.sync_copy(na_hbm, na_v)
    lanes = lax.iota(jnp.int32, 16)

    def issue(b, buf):
        rb = base + b * _CB
        pltpu.sync_copy(ki_hbm.at[pl.ds(rb * _K, _CB * _K)], idx_v.at[buf])
        pltpu.async_copy(h0_hbm.at[idx_v.at[buf]], nbr_v.at[buf], semn.at[buf])
        pltpu.async_copy(h0_hbm.at[pl.ds(rb, _CB)], self_v.at[buf],
                         sems.at[buf])

    def compute(b, buf):
        rb = base + b * _CB
        pltpu.make_async_copy(h0_hbm.at[idx_v.at[buf]], nbr_v.at[buf],
                              semn.at[buf]).wait()
        pltpu.make_async_copy(h0_hbm.at[pl.ds(rb, _CB)], self_v.at[buf],
                              sems.at[buf]).wait()
        for p in range(_CB // 2):
            accs = [jnp.zeros((16,), jnp.float32) for _ in range(16)]

            def cchunk(c8, accs):
                accs = list(accs)
                for cc in range(4):
                    off = (c8 * 4 + cc) * 16
                    s0 = self_v[buf, 2 * p, pl.ds(off, 16)]
                    s1 = self_v[buf, 2 * p + 1, pl.ds(off, 16)]
                    for k in range(_K):
                        accs[k] = accs[k] + s0 * nbr_v[buf, 2 * p * _K + k,
                                                       pl.ds(off, 16)]
                        accs[8 + k] = accs[8 + k] + s1 * nbr_v[
                            buf, (2 * p + 1) * _K + k, pl.ds(off, 16)]
                return tuple(accs)

            accs = lax.fori_loop(0, _D // 64, cchunk, tuple(accs))
            dvec = jnp.zeros((16,), jnp.float32)
            for j in range(16):
                dvec = jnp.where(lanes == j, jnp.sum(accs[j]), dvec)
            dots_v[pl.ds(p * 16, 16)] = dvec
        for g in range(_CB * _K // 16):
            idx16 = idx_v[buf, pl.ds(g * 16, 16)]
            nna_v[pl.ds(g * 16, 16)] = plsc.load_gather(na_v, [idx16])
        pltpu.sync_copy(dots_v, dots_hbm.at[pl.ds(rb * _K, _CB * _K)])
        pltpu.sync_copy(nna_v, nna_hbm.at[pl.ds(rb * _K, _CB * _K)])

    issue(0, 0)

    def pair(t, carry):
        issue(2 * t + 1, 1)
        compute(2 * t, 0)

        @pl.when(t < _NB // 2 - 1)
        def _():
            issue(2 * t + 2, 0)

        compute(2 * t + 1, 1)
        return carry

    lax.fori_loop(0, _NB // 2, pair, 0)


def _sc_dots(h0, ki_flat, na_flat):
    mesh = plsc.VectorSubcoreMesh(core_axis_name="c", subcore_axis_name="s")
    fn = functools.partial(
        pl.kernel,
        mesh=mesh,
        out_type=[
            jax.ShapeDtypeStruct((_N * _K,), jnp.float32),
            jax.ShapeDtypeStruct((_N * _K,), jnp.float32),
        ],
        scratch_types=[
            pltpu.VMEM((_N,), jnp.float32),
            pltpu.VMEM((2, _CB * _K), jnp.int32),
            pltpu.VMEM((2, _CB, _D), jnp.float32),
            pltpu.VMEM((2, _CB * _K, _D), jnp.float32),
            pltpu.VMEM((_CB * _K,), jnp.float32),
            pltpu.VMEM((_CB * _K,), jnp.float32),
            pltpu.SemaphoreType.DMA((2,)),
            pltpu.SemaphoreType.DMA((2,)),
        ],
        compiler_params=pltpu.CompilerParams(needs_layout_passes=False),
    )(_sc_dots_body)
    return fn(h0, ki_flat, na_flat)


def _epi_body(dots_ref, nna_ref, na_ref, kd_ref, w1a_ref, w1b_ref, b1_ref,
              w2r_ref, b2_ref, out_ref):
    na = na_ref[...]
    sim = dots_ref[...] / jnp.maximum(na * nna_ref[...], 1e-8)
    sw = jnp.exp(kd_ref[...] * (-1.0 / 0.05))
    sm = jnp.mean(sim, axis=1, keepdims=True)
    pm = jnp.mean(sw, axis=1, keepdims=True)
    h = jnp.maximum(sm * w1a_ref[...] + pm * w1b_ref[...] + b1_ref[...], 0.0)
    z = jnp.sum(h * w2r_ref[...], axis=1, keepdims=True) + b2_ref[...]
    out_ref[...] = 1.0 / (1.0 + jnp.exp(-z))


def _epi_call(dots, nna, na, kd, w1a, w1b, b1r, w2r, b2r):
    return pl.pallas_call(
        _epi_body,
        grid=(_N // _BC,),
        in_specs=[
            pl.BlockSpec((_BC, _K), lambda i: (i, 0)),
            pl.BlockSpec((_BC, _K), lambda i: (i, 0)),
            pl.BlockSpec((_BC, 1), lambda i: (i, 0)),
            pl.BlockSpec((_BC, _K), lambda i: (i, 0)),
            pl.BlockSpec((1, 64), lambda i: (0, 0)),
            pl.BlockSpec((1, 64), lambda i: (0, 0)),
            pl.BlockSpec((1, 64), lambda i: (0, 0)),
            pl.BlockSpec((1, 64), lambda i: (0, 0)),
            pl.BlockSpec((1, 1), lambda i: (0, 0)),
        ],
        out_specs=pl.BlockSpec((_BC, 1), lambda i: (i, 0)),
        out_shape=jax.ShapeDtypeStruct((_N, 1), jnp.float32),
    )(dots, nna, na, kd, w1a, w1b, b1r, w2r, b2r)


def kernel(h0, coords, W1, b1, W2, b2):
    ct = coords.T
    kd, ki, na = _knn_call(coords, ct, h0)
    dots_f, nna_f = _sc_dots(h0, ki.reshape(-1), na.reshape(-1))
    dots = dots_f.reshape(_N, _K)
    nna = nna_f.reshape(_N, _K)
    w1a = W1[:, 0].reshape(1, 64)
    w1b = W1[:, 1].reshape(1, 64)
    b1r = b1.reshape(1, 64)
    w2r = W2.reshape(1, 64)
    b2r = b2.reshape(1, 1)
    return _epi_call(dots, nna, na, kd, w1a, w1b, b1r, w2r, b2r)


# speedup vs baseline: 1.4836x; 1.0946x over previous
import functools

import jax
import jax.numpy as jnp
from jax import lax
from jax.experimental import pallas as pl
from jax.experimental.pallas import tpu as pltpu
from jax.experimental.pallas import tpu_sc as plsc

_N, _D, _K = 8192, 1024, 8
_NH = _N // 2
_M13 = 0x1FFF
_BIAS = 0x00800000
_BR = 256
_BC = 512
_NW = 32
_RPW = _NH // _NW
_CB = 4
_NB = _RPW // _CB


def _norms_body(h0_ref, na_ref):
    h = h0_ref[...]
    na_ref[...] = jnp.sqrt(jnp.sum(h * h, axis=1, keepdims=True))


def _norms_call(h0):
    return pl.pallas_call(
        _norms_body,
        grid=(_N // _BC,),
        in_specs=[pl.BlockSpec((_BC, _D), lambda i: (i, 0))],
        out_specs=pl.BlockSpec((_BC, 1), lambda i: (i, 0)),
        out_shape=jax.ShapeDtypeStruct((_N, 1), jnp.float32),
    )(h0)


def _knn_body(coords_ref, ct_ref, kd_ref, ki_ref):
    rx = coords_ref[:, 0:1]
    ry = coords_ref[:, 1:2]
    cx = ct_ref[0:1, :]
    cy = ct_ref[1:2, :]
    col = lax.broadcasted_iota(jnp.int32, (_BR, _N), 1)
    cross = jnp.dot(coords_ref[...], ct_ref[...],
                    preferred_element_type=jnp.float32)
    a2 = rx * rx + ry * ry
    b2 = cx * cx + cy * cy
    d2 = jnp.maximum(a2 + b2 - 2.0 * cross, 0.0)
    keyi = ((lax.bitcast_convert_type(d2, jnp.int32) & ~_M13) | col) + _BIAS
    key = lax.bitcast_convert_type(keyi, jnp.float32)
    big = lax.bitcast_convert_type(jnp.int32(0x7F000000), jnp.float32)
    col8 = lax.broadcasted_iota(jnp.int32, (_BR, _K), 1)
    kd = jnp.zeros((_BR, _K), jnp.float32)
    ki = jnp.zeros((_BR, _K), jnp.int32)
    m = jnp.min(key, axis=1, keepdims=True)
    for k in range(_K):
        m = jnp.min(jnp.where(key > m, key, big), axis=1, keepdims=True)
        mi = lax.bitcast_convert_type(m, jnp.int32) - _BIAS
        idx = mi & _M13
        d2t = lax.bitcast_convert_type(mi & ~_M13, jnp.float32)
        kd = jnp.where(col8 == k, jnp.sqrt(d2t), kd)
        ki = jnp.where(col8 == k, idx, ki)
    kd_ref[...] = kd
    ki_ref[...] = ki


def _knn_call(coords, ct, half):
    off = half * (_NH // _BR)
    return pl.pallas_call(
        _knn_body,
        grid=(_NH // _BR,),
        in_specs=[
            pl.BlockSpec((_BR, 2), lambda i: (i + off, 0)),
            pl.BlockSpec((2, _N), lambda i: (0, 0)),
        ],
        out_specs=[
            pl.BlockSpec((_BR, _K), lambda i: (i, 0)),
            pl.BlockSpec((_BR, _K), lambda i: (i, 0)),
        ],
        out_shape=[
            jax.ShapeDtypeStruct((_NH, _K), jnp.float32),
            jax.ShapeDtypeStruct((_NH, _K), jnp.int32),
        ],
    )(coords, ct)


def _make_sc_dots_body(base_row):
    def body(h0_hbm, ki_hbm, na_hbm, dots_hbm, nna_hbm,
             na_v, idx_v, self_v, nbr_v, dots_v, nna_v, sems, semn):
        wid = lax.axis_index("s") * 2 + lax.axis_index("c")
        base = wid * _RPW
        pltpu.sync_copy(na_hbm, na_v)
        lanes = lax.iota(jnp.int32, 16)

        def issue(b, buf):
            rb = base + b * _CB
            pltpu.sync_copy(ki_hbm.at[pl.ds(rb * _K, _CB * _K)],
                            idx_v.at[buf])
            pltpu.async_copy(h0_hbm.at[idx_v.at[buf]], nbr_v.at[buf],
                             semn.at[buf])
            pltpu.async_copy(h0_hbm.at[pl.ds(base_row + rb, _CB)],
                             self_v.at[buf], sems.at[buf])

        def compute(b, buf):
            rb = base + b * _CB
            pltpu.make_async_copy(h0_hbm.at[idx_v.at[buf]], nbr_v.at[buf],
                                  semn.at[buf]).wait()
            pltpu.make_async_copy(h0_hbm.at[pl.ds(base_row + rb, _CB)],
                                  self_v.at[buf], sems.at[buf]).wait()
            for p in range(_CB // 2):
                accs = [jnp.zeros((16,), jnp.float32) for _ in range(16)]

                def cchunk(c8, accs):
                    accs = list(accs)
                    for cc in range(4):
                        off = (c8 * 4 + cc) * 16
                        s0 = self_v[buf, 2 * p, pl.ds(off, 16)]
                        s1 = self_v[buf, 2 * p + 1, pl.ds(off, 16)]
                        for k in range(_K):
                            accs[k] = accs[k] + s0 * nbr_v[
                                buf, 2 * p * _K + k, pl.ds(off, 16)]
                            accs[8 + k] = accs[8 + k] + s1 * nbr_v[
                                buf, (2 * p + 1) * _K + k, pl.ds(off, 16)]
                    return tuple(accs)

                accs = lax.fori_loop(0, _D // 64, cchunk, tuple(accs))
                dvec = jnp.zeros((16,), jnp.float32)
                for j in range(16):
                    dvec = jnp.where(lanes == j, jnp.sum(accs[j]), dvec)
                dots_v[pl.ds(p * 16, 16)] = dvec
            for g in range(_CB * _K // 16):
                idx16 = idx_v[buf, pl.ds(g * 16, 16)]
                nna_v[pl.ds(g * 16, 16)] = plsc.load_gather(na_v, [idx16])
            pltpu.sync_copy(dots_v, dots_hbm.at[pl.ds(rb * _K, _CB * _K)])
            pltpu.sync_copy(nna_v, nna_hbm.at[pl.ds(rb * _K, _CB * _K)])

        issue(0, 0)

        def pair(t, carry):
            issue(2 * t + 1, 1)
            compute(2 * t, 0)

            @pl.when(t < _NB // 2 - 1)
            def _():
                issue(2 * t + 2, 0)

            compute(2 * t + 1, 1)
            return carry

        lax.fori_loop(0, _NB // 2, pair, 0)

    return body


def _sc_dots(h0, ki_flat, na_flat, base_row):
    mesh = plsc.VectorSubcoreMesh(core_axis_name="c", subcore_axis_name="s")
    fn = functools.partial(
        pl.kernel,
        mesh=mesh,
        out_type=[
            jax.ShapeDtypeStruct((_NH * _K,), jnp.float32),
            jax.ShapeDtypeStruct((_NH * _K,), jnp.float32),
        ],
        scratch_types=[
            pltpu.VMEM((_N,), jnp.float32),
            pltpu.VMEM((2, _CB * _K), jnp.int32),
            pltpu.VMEM((2, _CB, _D), jnp.float32),
            pltpu.VMEM((2, _CB * _K, _D), jnp.float32),
            pltpu.VMEM((_CB * _K,), jnp.float32),
            pltpu.VMEM((_CB * _K,), jnp.float32),
            pltpu.SemaphoreType.DMA((2,)),
            pltpu.SemaphoreType.DMA((2,)),
        ],
        compiler_params=pltpu.CompilerParams(needs_layout_passes=False),
    )(_make_sc_dots_body(base_row))
    return fn(h0, ki_flat, na_flat)


def _epi_body(dots_ref, nna_ref, na_ref, kd_ref, w1a_ref, w1b_ref, b1_ref,
              w2r_ref, b2_ref, out_ref):
    na = na_ref[...]
    sim = dots_ref[...] / jnp.maximum(na * nna_ref[...], 1e-8)
    sw = jnp.exp(kd_ref[...] * (-1.0 / 0.05))
    sm = jnp.mean(sim, axis=1, keepdims=True)
    pm = jnp.mean(sw, axis=1, keepdims=True)
    h = jnp.maximum(sm * w1a_ref[...] + pm * w1b_ref[...] + b1_ref[...], 0.0)
    z = jnp.sum(h * w2r_ref[...], axis=1, keepdims=True) + b2_ref[...]
    out_ref[...] = 1.0 / (1.0 + jnp.exp(-z))


def _epi_call(dots, nna, na, kd, w1a, w1b, b1r, w2r, b2r):
    return pl.pallas_call(
        _epi_body,
        grid=(_N // _BC,),
        in_specs=[
            pl.BlockSpec((_BC, _K), lambda i: (i, 0)),
            pl.BlockSpec((_BC, _K), lambda i: (i, 0)),
            pl.BlockSpec((_BC, 1), lambda i: (i, 0)),
            pl.BlockSpec((_BC, _K), lambda i: (i, 0)),
            pl.BlockSpec((1, 64), lambda i: (0, 0)),
            pl.BlockSpec((1, 64), lambda i: (0, 0)),
            pl.BlockSpec((1, 64), lambda i: (0, 0)),
            pl.BlockSpec((1, 64), lambda i: (0, 0)),
            pl.BlockSpec((1, 1), lambda i: (0, 0)),
        ],
        out_specs=pl.BlockSpec((_BC, 1), lambda i: (i, 0)),
        out_shape=jax.ShapeDtypeStruct((_N, 1), jnp.float32),
    )(dots, nna, na, kd, w1a, w1b, b1r, w2r, b2r)


def kernel(h0, coords, W1, b1, W2, b2):
    ct = coords.T
    na = _norms_call(h0)
    naf = na.reshape(-1)
    kd0, ki0 = _knn_call(coords, ct, 0)
    kd1, ki1 = _knn_call(coords, ct, 1)
    d0, n0 = _sc_dots(h0, ki0.reshape(-1), naf, 0)
    d1, n1 = _sc_dots(h0, ki1.reshape(-1), naf, _NH)
    dots = jnp.concatenate([d0, d1]).reshape(_N, _K)
    nna = jnp.concatenate([n0, n1]).reshape(_N, _K)
    kd = jnp.concatenate([kd0, kd1], axis=0)
    w1a = W1[:, 0].reshape(1, 64)
    w1b = W1[:, 1].reshape(1, 64)
    b1r = b1.reshape(1, 64)
    w2r = W2.reshape(1, 64)
    b2r = b2.reshape(1, 1)
    return _epi_call(dots, nna, na, kd, w1a, w1b, b1r, w2r, b2r)


# speedup vs baseline: 1.5539x; 1.0474x over previous
import functools

import jax
import jax.numpy as jnp
from jax import lax
from jax.experimental import pallas as pl
from jax.experimental.pallas import tpu as pltpu
from jax.experimental.pallas import tpu_sc as plsc

_N, _D, _K = 8192, 1024, 8
_NS = 4
_NH = _N // _NS
_M13 = 0x1FFF
_BIAS = 0x00800000
_BR = 256
_BC = 512
_NW = 32
_RPW = _NH // _NW
_CB = 4
_NB = _RPW // _CB


def _norms_body(h0_ref, na_ref):
    h = h0_ref[...]
    na_ref[...] = jnp.sqrt(jnp.sum(h * h, axis=1, keepdims=True))


def _norms_call(h0):
    return pl.pallas_call(
        _norms_body,
        grid=(_N // _BC,),
        in_specs=[pl.BlockSpec((_BC, _D), lambda i: (i, 0))],
        out_specs=pl.BlockSpec((_BC, 1), lambda i: (i, 0)),
        out_shape=jax.ShapeDtypeStruct((_N, 1), jnp.float32),
    )(h0)


def _knn_body(coords_ref, ct_ref, kd_ref, ki_ref):
    rx = coords_ref[:, 0:1]
    ry = coords_ref[:, 1:2]
    cx = ct_ref[0:1, :]
    cy = ct_ref[1:2, :]
    col = lax.broadcasted_iota(jnp.int32, (_BR, _N), 1)
    cross = jnp.dot(coords_ref[...], ct_ref[...],
                    preferred_element_type=jnp.float32)
    a2 = rx * rx + ry * ry
    b2 = cx * cx + cy * cy
    d2 = jnp.maximum(a2 + b2 - 2.0 * cross, 0.0)
    keyi = ((lax.bitcast_convert_type(d2, jnp.int32) & ~_M13) | col) + _BIAS
    key = lax.bitcast_convert_type(keyi, jnp.float32)
    big = lax.bitcast_convert_type(jnp.int32(0x7F000000), jnp.float32)
    col8 = lax.broadcasted_iota(jnp.int32, (_BR, _K), 1)
    kd = jnp.zeros((_BR, _K), jnp.float32)
    ki = jnp.zeros((_BR, _K), jnp.int32)
    m = jnp.min(key, axis=1, keepdims=True)
    for k in range(_K):
        m = jnp.min(jnp.where(key > m, key, big), axis=1, keepdims=True)
        mi = lax.bitcast_convert_type(m, jnp.int32) - _BIAS
        idx = mi & _M13
        d2t = lax.bitcast_convert_type(mi & ~_M13, jnp.float32)
        kd = jnp.where(col8 == k, jnp.sqrt(d2t), kd)
        ki = jnp.where(col8 == k, idx, ki)
    kd_ref[...] = kd
    ki_ref[...] = ki


def _knn_call(coords, ct, half):
    off = half * (_NH // _BR)
    return pl.pallas_call(
        _knn_body,
        grid=(_NH // _BR,),
        in_specs=[
            pl.BlockSpec((_BR, 2), lambda i: (i + off, 0)),
            pl.BlockSpec((2, _N), lambda i: (0, 0)),
        ],
        out_specs=[
            pl.BlockSpec((_BR, _K), lambda i: (i, 0)),
            pl.BlockSpec((_BR, _K), lambda i: (i, 0)),
        ],
        out_shape=[
            jax.ShapeDtypeStruct((_NH, _K), jnp.float32),
            jax.ShapeDtypeStruct((_NH, _K), jnp.int32),
        ],
    )(coords, ct)


def _make_sc_dots_body(base_row):
    def body(h0_hbm, ki_hbm, na_hbm, dots_hbm, nna_hbm,
             na_v, idx_v, self_v, nbr_v, dots_v, nna_v, sems, semn):
        wid = lax.axis_index("s") * 2 + lax.axis_index("c")
        base = wid * _RPW
        pltpu.sync_copy(na_hbm, na_v)
        lanes = lax.iota(jnp.int32, 16)

        def issue(b, buf):
            rb = base + b * _CB
            pltpu.sync_copy(ki_hbm.at[pl.ds(rb * _K, _CB * _K)],
                            idx_v.at[buf])
            pltpu.async_copy(h0_hbm.at[idx_v.at[buf]], nbr_v.at[buf],
                             semn.at[buf])
            pltpu.async_copy(h0_hbm.at[pl.ds(base_row + rb, _CB)],
                             self_v.at[buf], sems.at[buf])

        def compute(b, buf):
            rb = base + b * _CB
            pltpu.make_async_copy(h0_hbm.at[idx_v.at[buf]], nbr_v.at[buf],
                                  semn.at[buf]).wait()
            pltpu.make_async_copy(h0_hbm.at[pl.ds(base_row + rb, _CB)],
                                  self_v.at[buf], sems.at[buf]).wait()
            for p in range(_CB // 2):
                accs = [jnp.zeros((16,), jnp.float32) for _ in range(16)]

                def cchunk(c8, accs):
                    accs = list(accs)
                    for cc in range(4):
                        off = (c8 * 4 + cc) * 16
                        s0 = self_v[buf, 2 * p, pl.ds(off, 16)]
                        s1 = self_v[buf, 2 * p + 1, pl.ds(off, 16)]
                        for k in range(_K):
                            accs[k] = accs[k] + s0 * nbr_v[
                                buf, 2 * p * _K + k, pl.ds(off, 16)]
                            accs[8 + k] = accs[8 + k] + s1 * nbr_v[
                                buf, (2 * p + 1) * _K + k, pl.ds(off, 16)]
                    return tuple(accs)

                accs = lax.fori_loop(0, _D // 64, cchunk, tuple(accs))
                dvec = jnp.zeros((16,), jnp.float32)
                for j in range(16):
                    dvec = jnp.where(lanes == j, jnp.sum(accs[j]), dvec)
                dots_v[pl.ds(p * 16, 16)] = dvec
            for g in range(_CB * _K // 16):
                idx16 = idx_v[buf, pl.ds(g * 16, 16)]
                nna_v[pl.ds(g * 16, 16)] = plsc.load_gather(na_v, [idx16])
            pltpu.sync_copy(dots_v, dots_hbm.at[pl.ds(rb * _K, _CB * _K)])
            pltpu.sync_copy(nna_v, nna_hbm.at[pl.ds(rb * _K, _CB * _K)])

        issue(0, 0)

        def pair(t, carry):
            issue(2 * t + 1, 1)
            compute(2 * t, 0)

            @pl.when(t < _NB // 2 - 1)
            def _():
                issue(2 * t + 2, 0)

            compute(2 * t + 1, 1)
            return carry

        lax.fori_loop(0, _NB // 2, pair, 0)

    return body


def _sc_dots(h0, ki_flat, na_flat, base_row):
    mesh = plsc.VectorSubcoreMesh(core_axis_name="c", subcore_axis_name="s")
    fn = functools.partial(
        pl.kernel,
        mesh=mesh,
        out_type=[
            jax.ShapeDtypeStruct((_NH * _K,), jnp.float32),
            jax.ShapeDtypeStruct((_NH * _K,), jnp.float32),
        ],
        scratch_types=[
            pltpu.VMEM((_N,), jnp.float32),
            pltpu.VMEM((2, _CB * _K), jnp.int32),
            pltpu.VMEM((2, _CB, _D), jnp.float32),
            pltpu.VMEM((2, _CB * _K, _D), jnp.float32),
            pltpu.VMEM((_CB * _K,), jnp.float32),
            pltpu.VMEM((_CB * _K,), jnp.float32),
            pltpu.SemaphoreType.DMA((2,)),
            pltpu.SemaphoreType.DMA((2,)),
        ],
        compiler_params=pltpu.CompilerParams(needs_layout_passes=False),
    )(_make_sc_dots_body(base_row))
    return fn(h0, ki_flat, na_flat)


def _epi_body(dots_ref, nna_ref, na_ref, kd_ref, w1a_ref, w1b_ref, b1_ref,
              w2r_ref, b2_ref, out_ref):
    na = na_ref[...]
    sim = dots_ref[...] / jnp.maximum(na * nna_ref[...], 1e-8)
    sw = jnp.exp(kd_ref[...] * (-1.0 / 0.05))
    sm = jnp.mean(sim, axis=1, keepdims=True)
    pm = jnp.mean(sw, axis=1, keepdims=True)
    h = jnp.maximum(sm * w1a_ref[...] + pm * w1b_ref[...] + b1_ref[...], 0.0)
    z = jnp.sum(h * w2r_ref[...], axis=1, keepdims=True) + b2_ref[...]
    out_ref[...] = 1.0 / (1.0 + jnp.exp(-z))


def _epi_call(dots, nna, na, kd, w1a, w1b, b1r, w2r, b2r):
    return pl.pallas_call(
        _epi_body,
        grid=(_N // _BC,),
        in_specs=[
            pl.BlockSpec((_BC, _K), lambda i: (i, 0)),
            pl.BlockSpec((_BC, _K), lambda i: (i, 0)),
            pl.BlockSpec((_BC, 1), lambda i: (i, 0)),
            pl.BlockSpec((_BC, _K), lambda i: (i, 0)),
            pl.BlockSpec((1, 64), lambda i: (0, 0)),
            pl.BlockSpec((1, 64), lambda i: (0, 0)),
            pl.BlockSpec((1, 64), lambda i: (0, 0)),
            pl.BlockSpec((1, 64), lambda i: (0, 0)),
            pl.BlockSpec((1, 1), lambda i: (0, 0)),
        ],
        out_specs=pl.BlockSpec((_BC, 1), lambda i: (i, 0)),
        out_shape=jax.ShapeDtypeStruct((_N, 1), jnp.float32),
    )(dots, nna, na, kd, w1a, w1b, b1r, w2r, b2r)


def kernel(h0, coords, W1, b1, W2, b2):
    ct = coords.T
    na = _norms_call(h0)
    naf = na.reshape(-1)
    kds, dparts, nparts = [], [], []
    for q in range(_NS):
        kd_q, ki_q = _knn_call(coords, ct, q)
        d_q, n_q = _sc_dots(h0, ki_q.reshape(-1), naf, q * _NH)
        kds.append(kd_q)
        dparts.append(d_q)
        nparts.append(n_q)
    dots = jnp.concatenate(dparts).reshape(_N, _K)
    nna = jnp.concatenate(nparts).reshape(_N, _K)
    kd = jnp.concatenate(kds, axis=0)
    w1a = W1[:, 0].reshape(1, 64)
    w1b = W1[:, 1].reshape(1, 64)
    b1r = b1.reshape(1, 64)
    w2r = W2.reshape(1, 64)
    b2r = b2.reshape(1, 1)
    return _epi_call(dots, nna, na, kd, w1a, w1b, b1r, w2r, b2r)


# speedup vs baseline: 1.5960x; 1.0270x over previous
import functools

import jax
import jax.numpy as jnp
from jax import lax
from jax.experimental import pallas as pl
from jax.experimental.pallas import tpu as pltpu
from jax.experimental.pallas import tpu_sc as plsc

_N, _D, _K = 8192, 1024, 8
_NS = 4
_NH = _N // _NS
_M13 = 0x1FFF
_BIAS = 0x00800000
_BR = 256
_BC = 512
_NW = 32
_RPW = _NH // _NW
_CB = 4
_NB = _RPW // _CB


def _knn_body(coords_ref, ct_ref, kd_ref, ki_ref):
    rx = coords_ref[:, 0:1]
    ry = coords_ref[:, 1:2]
    cx = ct_ref[0:1, :]
    cy = ct_ref[1:2, :]
    col = lax.broadcasted_iota(jnp.int32, (_BR, _N), 1)
    cross = jnp.dot(coords_ref[...], ct_ref[...],
                    preferred_element_type=jnp.float32)
    a2 = rx * rx + ry * ry
    b2 = cx * cx + cy * cy
    d2 = jnp.maximum(a2 + b2 - 2.0 * cross, 0.0)
    keyi = ((lax.bitcast_convert_type(d2, jnp.int32) & ~_M13) | col) + _BIAS
    key = lax.bitcast_convert_type(keyi, jnp.float32)
    big = lax.bitcast_convert_type(jnp.int32(0x7F000000), jnp.float32)
    col8 = lax.broadcasted_iota(jnp.int32, (_BR, _K), 1)
    kd = jnp.zeros((_BR, _K), jnp.float32)
    ki = jnp.zeros((_BR, _K), jnp.int32)
    m = jnp.min(key, axis=1, keepdims=True)
    for k in range(_K):
        m = jnp.min(jnp.where(key > m, key, big), axis=1, keepdims=True)
        mi = lax.bitcast_convert_type(m, jnp.int32) - _BIAS
        idx = mi & _M13
        d2t = lax.bitcast_convert_type(mi & ~_M13, jnp.float32)
        kd = jnp.where(col8 == k, jnp.sqrt(d2t), kd)
        ki = jnp.where(col8 == k, idx, ki)
    kd_ref[...] = kd
    ki_ref[...] = ki


def _knn_norms_body(coords_ref, ct_ref, h0_ref, kd_ref, ki_ref, na_ref):
    _knn_body(coords_ref, ct_ref, kd_ref, ki_ref)
    h = h0_ref[...]
    na_ref[...] = jnp.sqrt(jnp.sum(h * h, axis=1, keepdims=True))


def _knn_call(coords, ct, half, h0=None):
    off = half * (_NH // _BR)
    nsteps = _NH // _BR
    if h0 is None:
        return pl.pallas_call(
            _knn_body,
            grid=(nsteps,),
            in_specs=[
                pl.BlockSpec((_BR, 2), lambda i: (i + off, 0)),
                pl.BlockSpec((2, _N), lambda i: (0, 0)),
            ],
            out_specs=[
                pl.BlockSpec((_BR, _K), lambda i: (i, 0)),
                pl.BlockSpec((_BR, _K), lambda i: (i, 0)),
            ],
            out_shape=[
                jax.ShapeDtypeStruct((_NH, _K), jnp.float32),
                jax.ShapeDtypeStruct((_NH, _K), jnp.int32),
            ],
        )(coords, ct)
    nb = _N // nsteps
    return pl.pallas_call(
        _knn_norms_body,
        grid=(nsteps,),
        in_specs=[
            pl.BlockSpec((_BR, 2), lambda i: (i + off, 0)),
            pl.BlockSpec((2, _N), lambda i: (0, 0)),
            pl.BlockSpec((nb, _D), lambda i: (i, 0)),
        ],
        out_specs=[
            pl.BlockSpec((_BR, _K), lambda i: (i, 0)),
            pl.BlockSpec((_BR, _K), lambda i: (i, 0)),
            pl.BlockSpec((nb, 1), lambda i: (i, 0)),
        ],
        out_shape=[
            jax.ShapeDtypeStruct((_NH, _K), jnp.float32),
            jax.ShapeDtypeStruct((_NH, _K), jnp.int32),
            jax.ShapeDtypeStruct((_N, 1), jnp.float32),
        ],
    )(coords, ct, h0)


def _make_sc_dots_body(base_row):
    def body(h0_hbm, ki_hbm, na_hbm, dots_hbm, nna_hbm,
             na_v, idx_v, self_v, nbr_v, dots_v, nna_v, sems, semn):
        wid = lax.axis_index("s") * 2 + lax.axis_index("c")
        base = wid * _RPW
        pltpu.sync_copy(na_hbm, na_v)
        lanes = lax.iota(jnp.int32, 16)

        def issue(b, buf):
            rb = base + b * _CB
            pltpu.sync_copy(ki_hbm.at[pl.ds(rb * _K, _CB * _K)],
                            idx_v.at[buf])
            pltpu.async_copy(h0_hbm.at[idx_v.at[buf]], nbr_v.at[buf],
                             semn.at[buf])
            pltpu.async_copy(h0_hbm.at[pl.ds(base_row + rb, _CB)],
                             self_v.at[buf], sems.at[buf])

        def compute(b, buf):
            rb = base + b * _CB
            pltpu.make_async_copy(h0_hbm.at[idx_v.at[buf]], nbr_v.at[buf],
                                  semn.at[buf]).wait()
            pltpu.make_async_copy(h0_hbm.at[pl.ds(base_row + rb, _CB)],
                                  self_v.at[buf], sems.at[buf]).wait()
            for p in range(_CB // 2):
                accs = [jnp.zeros((16,), jnp.float32) for _ in range(16)]

                def cchunk(c8, accs):
                    accs = list(accs)
                    for cc in range(4):
                        off = (c8 * 4 + cc) * 16
                        s0 = self_v[buf, 2 * p, pl.ds(off, 16)]
                        s1 = self_v[buf, 2 * p + 1, pl.ds(off, 16)]
                        for k in range(_K):
                            accs[k] = accs[k] + s0 * nbr_v[
                                buf, 2 * p * _K + k, pl.ds(off, 16)]
                            accs[8 + k] = accs[8 + k] + s1 * nbr_v[
                                buf, (2 * p + 1) * _K + k, pl.ds(off, 16)]
                    return tuple(accs)

                accs = lax.fori_loop(0, _D // 64, cchunk, tuple(accs))
                dvec = jnp.zeros((16,), jnp.float32)
                for j in range(16):
                    dvec = jnp.where(lanes == j, jnp.sum(accs[j]), dvec)
                dots_v[pl.ds(p * 16, 16)] = dvec
            for g in range(_CB * _K // 16):
                idx16 = idx_v[buf, pl.ds(g * 16, 16)]
                nna_v[pl.ds(g * 16, 16)] = plsc.load_gather(na_v, [idx16])
            pltpu.sync_copy(dots_v, dots_hbm.at[pl.ds(rb * _K, _CB * _K)])
            pltpu.sync_copy(nna_v, nna_hbm.at[pl.ds(rb * _K, _CB * _K)])

        issue(0, 0)

        def pair(t, carry):
            issue(2 * t + 1, 1)
            compute(2 * t, 0)

            @pl.when(t < _NB // 2 - 1)
            def _():
                issue(2 * t + 2, 0)

            compute(2 * t + 1, 1)
            return carry

        lax.fori_loop(0, _NB // 2, pair, 0)

    return body


def _sc_dots(h0, ki_flat, na_flat, base_row):
    mesh = plsc.VectorSubcoreMesh(core_axis_name="c", subcore_axis_name="s")
    fn = functools.partial(
        pl.kernel,
        mesh=mesh,
        out_type=[
            jax.ShapeDtypeStruct((_NH * _K,), jnp.float32),
            jax.ShapeDtypeStruct((_NH * _K,), jnp.float32),
        ],
        scratch_types=[
            pltpu.VMEM((_N,), jnp.float32),
            pltpu.VMEM((2, _CB * _K), jnp.int32),
            pltpu.VMEM((2, _CB, _D), jnp.float32),
            pltpu.VMEM((2, _CB * _K, _D), jnp.float32),
            pltpu.VMEM((_CB * _K,), jnp.float32),
            pltpu.VMEM((_CB * _K,), jnp.float32),
            pltpu.SemaphoreType.DMA((2,)),
            pltpu.SemaphoreType.DMA((2,)),
        ],
        compiler_params=pltpu.CompilerParams(needs_layout_passes=False),
    )(_make_sc_dots_body(base_row))
    return fn(h0, ki_flat, na_flat)


def _epi_body(dots_ref, nna_ref, na_ref, kd_ref, w1a_ref, w1b_ref, b1_ref,
              w2r_ref, b2_ref, out_ref):
    na = na_ref[...]
    sim = dots_ref[...] / jnp.maximum(na * nna_ref[...], 1e-8)
    sw = jnp.exp(kd_ref[...] * (-1.0 / 0.05))
    sm = jnp.mean(sim, axis=1, keepdims=True)
    pm = jnp.mean(sw, axis=1, keepdims=True)
    h = jnp.maximum(sm * w1a_ref[...] + pm * w1b_ref[...] + b1_ref[...], 0.0)
    z = jnp.sum(h * w2r_ref[...], axis=1, keepdims=True) + b2_ref[...]
    out_ref[...] = 1.0 / (1.0 + jnp.exp(-z))


def _epi_call(dots, nna, na, kd, w1a, w1b, b1r, w2r, b2r):
    return pl.pallas_call(
        _epi_body,
        grid=(_N // _BC,),
        in_specs=[
            pl.BlockSpec((_BC, _K), lambda i: (i, 0)),
            pl.BlockSpec((_BC, _K), lambda i: (i, 0)),
            pl.BlockSpec((_BC, 1), lambda i: (i, 0)),
            pl.BlockSpec((_BC, _K), lambda i: (i, 0)),
            pl.BlockSpec((1, 64), lambda i: (0, 0)),
            pl.BlockSpec((1, 64), lambda i: (0, 0)),
            pl.BlockSpec((1, 64), lambda i: (0, 0)),
            pl.BlockSpec((1, 64), lambda i: (0, 0)),
            pl.BlockSpec((1, 1), lambda i: (0, 0)),
        ],
        out_specs=pl.BlockSpec((_BC, 1), lambda i: (i, 0)),
        out_shape=jax.ShapeDtypeStruct((_N, 1), jnp.float32),
    )(dots, nna, na, kd, w1a, w1b, b1r, w2r, b2r)


def kernel(h0, coords, W1, b1, W2, b2):
    ct = coords.T
    kd0, ki0, na = _knn_call(coords, ct, 0, h0=h0)
    naf = na.reshape(-1)
    d0, n0 = _sc_dots(h0, ki0.reshape(-1), naf, 0)
    kds, dparts, nparts = [kd0], [d0], [n0]
    for q in range(1, _NS):
        kd_q, ki_q = _knn_call(coords, ct, q)
        d_q, n_q = _sc_dots(h0, ki_q.reshape(-1), naf, q * _NH)
        kds.append(kd_q)
        dparts.append(d_q)
        nparts.append(n_q)
    dots = jnp.concatenate(dparts).reshape(_N, _K)
    nna = jnp.concatenate(nparts).reshape(_N, _K)
    kd = jnp.concatenate(kds, axis=0)
    w1a = W1[:, 0].reshape(1, 64)
    w1b = W1[:, 1].reshape(1, 64)
    b1r = b1.reshape(1, 64)
    w2r = W2.reshape(1, 64)
    b2r = b2.reshape(1, 1)
    return _epi_call(dots, nna, na, kd, w1a, w1b, b1r, w2r, b2r)
